# Initial kernel scaffold; baseline (speedup 1.0000x reference)
import jax
import jax.numpy as jnp
from jax.experimental import pallas as pl


def kernel(x, edge_weight, rs0_Wr, rs0_Ws, rs0_gamma, rs0_beta, rs1_Wr, rs1_Ws, rs1_gamma, rs1_beta, sc0_Wq, sc0_Wk, sc1_Wq, sc1_Wk, gn0_Wr, gn0_Ws, gn1_Wr, gn1_Ws, node_in, node_out, relation):
    raise NotImplementedError("write your pallas kernel here")


# speedup vs baseline: 4.0430x; 4.0430x over previous
import functools
import jax
import jax.numpy as jnp
from jax import lax
from jax.experimental import pallas as pl

N = 2048
E = 32768
R = 4
D = 64
WINDOW = 64
K = 16
NB = N // 128
BW = 256
PAD = 2176

_HIGH = jax.lax.Precision.DEFAULT


def _dot(a, b, dims):
    return lax.dot_general(a, b, (dims, ((), ())),
                           preferred_element_type=jnp.float32,
                           precision=_HIGH)


def _relstack_body(rewired, g_ref, bnd_ref, x_ref, wr_ref, ws_ref,
                   gam_ref, bet_ref, out_ref):
    xws = _dot(x_ref[...], ws_ref[...], ((1,), (0,)))
    gam = gam_ref[...]
    bet = bet_ref[...]
    for r in range(R):
        if rewired:
            if r < 2:
                agg = g_ref[r + 2] + bnd_ref[r]
            else:
                agg = g_ref[r - 2]
        else:
            agg = g_ref[r]
        h = _dot(agg, wr_ref[r], ((1,), (0,))) + xws
        mean = jnp.mean(h, axis=0, keepdims=True)
        var = jnp.mean((h - mean) * (h - mean), axis=0, keepdims=True)
        h = (h - mean) / jnp.sqrt(var + 1e-5) * gam + bet
        out_ref[r] = jnp.maximum(h, 0.0)


def _relstack(g, bnd, xin, wr, ws, gamma, beta, rewired):
    din = xin.shape[1]
    return pl.pallas_call(
        functools.partial(_relstack_body, rewired),
        out_shape=jax.ShapeDtypeStruct((R, N, D), jnp.float32),
    )(g, bnd, xin, wr, ws, gamma.reshape(1, D), beta.reshape(1, D))


def _gearnet_body(g_ref, bnd_ref, x_ref, wr_ref, ws_ref, out_ref, gsum_ref):
    h = _dot(x_ref[...], ws_ref[...], ((1,), (0,)))
    for r in range(R):
        if r < 2:
            agg = g_ref[r + 2] + bnd_ref[r]
        else:
            agg = g_ref[r - 2]
        h = h + _dot(agg, wr_ref[r], ((1,), (0,)))
    h = jnp.maximum(h, 0.0)
    out_ref[...] = h
    gsum_ref[...] = jnp.sum(h, axis=0, keepdims=True)


def _gearnet(g, bnd, xin, wr, ws):
    return pl.pallas_call(
        _gearnet_body,
        out_shape=[jax.ShapeDtypeStruct((N, D), jnp.float32),
                   jax.ShapeDtypeStruct((1, D), jnp.float32)],
    )(g, bnd, xin, wr, ws)


def _qk_body(rel_ref, wq_ref, wk_ref, q_ref, k_ref):
    wq = wq_ref[...]
    wk = wk_ref[...]
    for r in range(2):
        ai = rel_ref[r + 2]
        q_ref[r] = _dot(ai, wq, ((1,), (0,)))
        kf = _dot(ai, wk, ((1,), (0,)))
        k_ref[r, pl.ds(0, WINDOW), :] = jnp.zeros((WINDOW, D), jnp.float32)
        k_ref[r, pl.ds(WINDOW, N), :] = kf
        k_ref[r, pl.ds(WINDOW + N, PAD - N - WINDOW), :] = (
            jnp.zeros((PAD - N - WINDOW, D), jnp.float32))


def _qk(rel_out, wq, wk):
    return pl.pallas_call(
        _qk_body,
        out_shape=[jax.ShapeDtypeStruct((2, N, D), jnp.float32),
                   jax.ShapeDtypeStruct((2, PAD, D), jnp.float32)],
    )(rel_out, wq, wk)


def _attn_body(q_ref, k_ref, occ_ref, hm_ref):
    b = pl.program_id(1)
    qb = q_ref[0]
    ks = k_ref[0, pl.ds(pl.multiple_of(128 * b, 128), BW), :]
    s = _dot(qb, ks, ((1,), (1,))) * 0.125
    ii = lax.broadcasted_iota(jnp.int32, (128, BW), 0)
    cc = lax.broadcasted_iota(jnp.int32, (128, BW), 1)
    j = cc + 128 * b - WINDOW
    valid = (cc >= ii) & (cc <= ii + 2 * WINDOW) & (j >= 0) & (j < N)
    ml = jnp.where(valid, s, -1e30)
    m = jnp.max(ml, axis=1, keepdims=True)
    e = jnp.where(valid, jnp.exp(ml - m), 0.0)
    p = e / jnp.sum(e, axis=1, keepdims=True)
    t0 = jnp.max(p, axis=1, keepdims=True)
    def step(_, t):
        return jnp.max(jnp.where(p < t, p, -1.0), axis=1, keepdims=True)
    t = lax.fori_loop(0, K - 1, step, t0)
    hard = jnp.where(p >= t, p, 0.0)
    hm_ref[0, 0] = jnp.where(occ_ref[0, 0] > 0.0, 0.0, hard)


def _attn(qf, kbuf, occ):
    return pl.pallas_call(
        _attn_body,
        grid=(2, NB),
        in_specs=[
            pl.BlockSpec((1, 128, D), lambda r, b: (r, b, 0)),
            pl.BlockSpec((1, PAD, D), lambda r, b: (r, 0, 0)),
            pl.BlockSpec((1, 1, 128, BW), lambda r, b: (r, b, 0, 0)),
        ],
        out_specs=pl.BlockSpec((1, 1, 128, BW), lambda r, b: (r, b, 0, 0)),
        out_shape=jax.ShapeDtypeStruct((2, NB, 128, BW), jnp.float32),
    )(qf, kbuf, occ)


def _banded_body(hm_ref, x_ref, out_ref, acc_ref):
    b = pl.program_id(1)

    @pl.when(b == 0)
    def _():
        acc_ref[...] = jnp.zeros_like(acc_ref)

    xb = x_ref[pl.ds(pl.multiple_of(128 * b, 128), 128), :]
    c = _dot(hm_ref[0, 0], xb, ((0,), (0,)))
    off = pl.multiple_of(128 * b, 128)
    acc_ref[pl.ds(off, BW), :] = acc_ref[pl.ds(off, BW), :] + c

    @pl.when(b == NB - 1)
    def _():
        out_ref[0] = acc_ref[pl.ds(WINDOW, N), :]


def _banded(hm, xin):
    din = xin.shape[1]
    from jax.experimental.pallas import tpu as pltpu
    return pl.pallas_call(
        _banded_body,
        grid=(2, NB),
        in_specs=[
            pl.BlockSpec((1, 1, 128, BW), lambda r, b: (r, b, 0, 0)),
            pl.BlockSpec((N, din), lambda r, b: (0, 0)),
        ],
        out_specs=pl.BlockSpec((1, N, din), lambda r, b: (r, 0, 0)),
        out_shape=jax.ShapeDtypeStruct((2, N, din), jnp.float32),
        scratch_shapes=[pltpu.VMEM((PAD, din), jnp.float32)],
    )(hm, xin)


def _segsum(xin, node_in, dest, edge_weight):
    din = xin.shape[1]
    acc = jnp.zeros((R * N, din), jnp.float32)
    acc = acc.at[dest].add(xin[node_in] * edge_weight[:, None])
    return acc.reshape(R, N, din)


def _build_occ(node_in, node_out, relation):
    i = node_in.astype(jnp.int32)
    j = node_out.astype(jnp.int32)
    r = relation.astype(jnp.int32)
    b = i // 128
    ii = i % 128
    c = j - 128 * b + WINDOW
    ok = (r >= 2) & (c >= 0) & (c < BW)
    flat = (((r - 2) * NB + b) * 128 + ii) * BW + c
    flat = jnp.where(ok, flat, 2 * NB * 128 * BW)
    occ = jnp.zeros((2 * NB * 128 * BW + 1,), jnp.float32).at[flat].add(1.0)
    return occ[:-1].reshape(2, NB, 128, BW)


def kernel(x, edge_weight, rs0_Wr, rs0_Ws, rs0_gamma, rs0_beta,
           rs1_Wr, rs1_Ws, rs1_gamma, rs1_beta,
           sc0_Wq, sc0_Wk, sc1_Wq, sc1_Wk,
           gn0_Wr, gn0_Ws, gn1_Wr, gn1_Ws,
           node_in, node_out, relation):
    node_in = node_in.astype(jnp.int32)
    node_out = node_out.astype(jnp.int32)
    relation = relation.astype(jnp.int32)
    dest = relation * N + node_out

    occ = _build_occ(node_in, node_out, relation)
    g_x = _segsum(x, node_in, dest, edge_weight)

    rel0 = _relstack(g_x, jnp.zeros((2, N, D), jnp.float32), x,
                     rs0_Wr, rs0_Ws, rs0_gamma, rs0_beta, rewired=False)
    q0, k0 = _qk(rel0, sc0_Wq, sc0_Wk)
    hm0 = _attn(q0, k0, occ)
    bnd0x = _banded(hm0, x)
    h0, _ = _gearnet(g_x, bnd0x, x, gn0_Wr, gn0_Ws)

    sl = jnp.concatenate([h0, rel0.reshape(N, R * D)], axis=-1)
    g_sl = _segsum(sl, node_in, dest, edge_weight)
    g_h = _segsum(h0, node_in, dest, edge_weight)
    bnd0sl = _banded(hm0, sl)

    rel1 = _relstack(g_sl, bnd0sl, sl,
                     rs1_Wr, rs1_Ws, rs1_gamma, rs1_beta, rewired=True)
    q1, k1 = _qk(rel1, sc1_Wq, sc1_Wk)
    hm1 = _attn(q1, k1, occ)
    bnd1h = _banded(hm1, h0)
    hidden, gsum = _gearnet(g_h, bnd1h, h0, gn1_Wr, gn1_Ws)

    return gsum.reshape(D), hidden


# speedup vs baseline: 8.2379x; 2.0376x over previous
import functools
import jax
import jax.numpy as jnp
from jax import lax
from jax.experimental import pallas as pl

N = 2048
E = 32768
R = 4
D = 64
WINDOW = 64
K = 16
NB = N // 128
BW = 256
PAD = 2176

_HIGH = jax.lax.Precision.DEFAULT


def _dot(a, b, dims):
    return lax.dot_general(a, b, (dims, ((), ())),
                           preferred_element_type=jnp.float32,
                           precision=_HIGH)


def _relstack_body(rewired, g_ref, bnd_ref, x_ref, wr_ref, ws_ref,
                   gam_ref, bet_ref, out_ref):
    xws = _dot(x_ref[...], ws_ref[...], ((1,), (0,)))
    gam = gam_ref[...]
    bet = bet_ref[...]
    for r in range(R):
        if rewired:
            if r < 2:
                agg = g_ref[r + 2] + bnd_ref[r]
            else:
                agg = g_ref[r - 2]
        else:
            agg = g_ref[r]
        h = _dot(agg, wr_ref[r], ((1,), (0,))) + xws
        mean = jnp.mean(h, axis=0, keepdims=True)
        var = jnp.mean((h - mean) * (h - mean), axis=0, keepdims=True)
        h = (h - mean) / jnp.sqrt(var + 1e-5) * gam + bet
        out_ref[r] = jnp.maximum(h, 0.0)


def _relstack(g, bnd, xin, wr, ws, gamma, beta, rewired):
    din = xin.shape[1]
    return pl.pallas_call(
        functools.partial(_relstack_body, rewired),
        out_shape=jax.ShapeDtypeStruct((R, N, D), jnp.float32),
    )(g, bnd, xin, wr, ws, gamma.reshape(1, D), beta.reshape(1, D))


def _gearnet_body(g_ref, bnd_ref, x_ref, wr_ref, ws_ref, out_ref, gsum_ref):
    h = _dot(x_ref[...], ws_ref[...], ((1,), (0,)))
    for r in range(R):
        if r < 2:
            agg = g_ref[r + 2] + bnd_ref[r]
        else:
            agg = g_ref[r - 2]
        h = h + _dot(agg, wr_ref[r], ((1,), (0,)))
    h = jnp.maximum(h, 0.0)
    out_ref[...] = h
    gsum_ref[...] = jnp.sum(h, axis=0, keepdims=True)


def _gearnet(g, bnd, xin, wr, ws):
    return pl.pallas_call(
        _gearnet_body,
        out_shape=[jax.ShapeDtypeStruct((N, D), jnp.float32),
                   jax.ShapeDtypeStruct((1, D), jnp.float32)],
    )(g, bnd, xin, wr, ws)


def _qk_body(rel_ref, wq_ref, wk_ref, q_ref, k_ref):
    wq = wq_ref[...]
    wk = wk_ref[...]
    for r in range(2):
        ai = rel_ref[r + 2]
        q_ref[r] = _dot(ai, wq, ((1,), (0,)))
        kf = _dot(ai, wk, ((1,), (0,)))
        k_ref[r, pl.ds(0, WINDOW), :] = jnp.zeros((WINDOW, D), jnp.float32)
        k_ref[r, pl.ds(WINDOW, N), :] = kf
        k_ref[r, pl.ds(WINDOW + N, PAD - N - WINDOW), :] = (
            jnp.zeros((PAD - N - WINDOW, D), jnp.float32))


def _qk(rel_out, wq, wk):
    return pl.pallas_call(
        _qk_body,
        out_shape=[jax.ShapeDtypeStruct((2, N, D), jnp.float32),
                   jax.ShapeDtypeStruct((2, PAD, D), jnp.float32)],
    )(rel_out, wq, wk)


def _attn_body(q_ref, k_ref, occ_ref, hm_ref):
    b = pl.program_id(1)
    qb = q_ref[0]
    ks = k_ref[0, pl.ds(pl.multiple_of(128 * b, 128), BW), :]
    s = _dot(qb, ks, ((1,), (1,))) * 0.125
    ii = lax.broadcasted_iota(jnp.int32, (128, BW), 0)
    cc = lax.broadcasted_iota(jnp.int32, (128, BW), 1)
    j = cc + 128 * b - WINDOW
    valid = (cc >= ii) & (cc <= ii + 2 * WINDOW) & (j >= 0) & (j < N)
    ml = jnp.where(valid, s, -1e30)
    m = jnp.max(ml, axis=1, keepdims=True)
    e = jnp.where(valid, jnp.exp(ml - m), 0.0)
    p = e / jnp.sum(e, axis=1, keepdims=True)
    t0 = jnp.max(p, axis=1, keepdims=True)
    def step(_, t):
        return jnp.max(jnp.where(p < t, p, -1.0), axis=1, keepdims=True)
    t = lax.fori_loop(0, K - 1, step, t0)
    hard = jnp.where(p >= t, p, 0.0)
    hm_ref[0, 0] = jnp.where(occ_ref[0, 0] > 0.0, 0.0, hard)


def _attn(qf, kbuf, occ):
    return pl.pallas_call(
        _attn_body,
        grid=(2, NB),
        in_specs=[
            pl.BlockSpec((1, 128, D), lambda r, b: (r, b, 0)),
            pl.BlockSpec((1, PAD, D), lambda r, b: (r, 0, 0)),
            pl.BlockSpec((1, 1, 128, BW), lambda r, b: (r, b, 0, 0)),
        ],
        out_specs=pl.BlockSpec((1, 1, 128, BW), lambda r, b: (r, b, 0, 0)),
        out_shape=jax.ShapeDtypeStruct((2, NB, 128, BW), jnp.float32),
    )(qf, kbuf, occ)


def _banded_body(hm_ref, x_ref, out_ref, acc_ref):
    b = pl.program_id(1)

    @pl.when(b == 0)
    def _():
        acc_ref[...] = jnp.zeros_like(acc_ref)

    xb = x_ref[pl.ds(pl.multiple_of(128 * b, 128), 128), :]
    c = _dot(hm_ref[0, 0], xb, ((0,), (0,)))
    off = pl.multiple_of(128 * b, 128)
    acc_ref[pl.ds(off, BW), :] = acc_ref[pl.ds(off, BW), :] + c

    @pl.when(b == NB - 1)
    def _():
        out_ref[0] = acc_ref[pl.ds(WINDOW, N), :]


def _banded(hm, xin):
    din = xin.shape[1]
    from jax.experimental.pallas import tpu as pltpu
    return pl.pallas_call(
        _banded_body,
        grid=(2, NB),
        in_specs=[
            pl.BlockSpec((1, 1, 128, BW), lambda r, b: (r, b, 0, 0)),
            pl.BlockSpec((N, din), lambda r, b: (0, 0)),
        ],
        out_specs=pl.BlockSpec((1, N, din), lambda r, b: (r, 0, 0)),
        out_shape=jax.ShapeDtypeStruct((2, N, din), jnp.float32),
        scratch_shapes=[pltpu.VMEM((PAD, din), jnp.float32)],
    )(hm, xin)


NW = 32
GRP = 128
NG = E // (NW * GRP)


def _sc_segsum_call(xin, src3, dst3, zeros):
    from jax.experimental.pallas import tpu as pltpu
    from jax.experimental.pallas import tpu_sc as plsc
    din = xin.shape[1]
    mesh = plsc.VectorSubcoreMesh(core_axis_name="c", subcore_axis_name="s")

    @functools.partial(
        pl.kernel, mesh=mesh,
        out_type=jax.ShapeDtypeStruct((2, R * N, din), jnp.float32),
        scratch_types=[
            pltpu.VMEM((NG, GRP), jnp.int32),
            pltpu.VMEM((NG, GRP), jnp.int32),
            pltpu.VMEM((GRP, din), jnp.float32),
            pltpu.VMEM_SHARED((R * N, din), jnp.float32),
            pltpu.SemaphoreType.DMA,
        ],
    )
    def k(x_hbm, src_hbm, dst_hbm, z_hbm, out_hbm, src_v, dst_v, rows_v,
          shared, sem):
        cid = lax.axis_index("c")
        sid = lax.axis_index("s")
        wid = sid * 2 + cid

        @pl.when(sid == 0)
        def _():
            pltpu.sync_copy(z_hbm, shared)

        pltpu.sync_copy(src_hbm.at[wid], src_v)
        pltpu.sync_copy(dst_hbm.at[wid], dst_v)
        plsc.subcore_barrier()
        for g in range(NG):
            pltpu.async_copy(x_hbm.at[src_v.at[g]], rows_v, sem).wait()
            pltpu.sync_copy(rows_v, shared.at[dst_v.at[g]], add=True)
        plsc.subcore_barrier()
        rows_per = (R * N) // 16
        pltpu.sync_copy(shared.at[pl.ds(sid * rows_per, rows_per)],
                        out_hbm.at[cid, pl.ds(sid * rows_per, rows_per)])

    return k(xin, src3, dst3, zeros)


def _segsum(xin, node_in, dest, edge_weight):
    din = xin.shape[1]
    cw = 128
    dpad = ((din + cw - 1) // cw) * cw
    if dpad != din:
        xin = jnp.pad(xin, ((0, 0), (0, dpad - din)))
    src3 = node_in.reshape(NW, NG, GRP)
    dst3 = dest.reshape(NW, NG, GRP)
    zeros = jnp.zeros((R * N, cw), jnp.float32)
    parts = []
    for c0 in range(0, dpad, cw):
        part = _sc_segsum_call(xin[:, c0:c0 + cw], src3, dst3, zeros)
        parts.append(part[0] + part[1])
    acc = jnp.concatenate(parts, axis=1) if len(parts) > 1 else parts[0]
    return acc[:, :din].reshape(R, N, din)


def _build_occ(node_in, node_out, relation):
    i = node_in.astype(jnp.int32)
    j = node_out.astype(jnp.int32)
    r = relation.astype(jnp.int32)
    b = i // 128
    ii = i % 128
    c = j - 128 * b + WINDOW
    ok = (r >= 2) & (c >= 0) & (c < BW)
    flat = (((r - 2) * NB + b) * 128 + ii) * BW + c
    flat = jnp.where(ok, flat, 2 * NB * 128 * BW)
    occ = jnp.zeros((2 * NB * 128 * BW + 1,), jnp.float32).at[flat].add(1.0)
    return occ[:-1].reshape(2, NB, 128, BW)


def kernel(x, edge_weight, rs0_Wr, rs0_Ws, rs0_gamma, rs0_beta,
           rs1_Wr, rs1_Ws, rs1_gamma, rs1_beta,
           sc0_Wq, sc0_Wk, sc1_Wq, sc1_Wk,
           gn0_Wr, gn0_Ws, gn1_Wr, gn1_Ws,
           node_in, node_out, relation):
    node_in = node_in.astype(jnp.int32)
    node_out = node_out.astype(jnp.int32)
    relation = relation.astype(jnp.int32)
    dest = relation * N + node_out

    occ = _build_occ(node_in, node_out, relation)
    g_x = _segsum(x, node_in, dest, edge_weight)

    rel0 = _relstack(g_x, jnp.zeros((2, N, D), jnp.float32), x,
                     rs0_Wr, rs0_Ws, rs0_gamma, rs0_beta, rewired=False)
    q0, k0 = _qk(rel0, sc0_Wq, sc0_Wk)
    hm0 = _attn(q0, k0, occ)
    bnd0x = _banded(hm0, x)
    h0, _ = _gearnet(g_x, bnd0x, x, gn0_Wr, gn0_Ws)

    sl = jnp.concatenate([h0, rel0.reshape(N, R * D)], axis=-1)
    g_sl = _segsum(sl, node_in, dest, edge_weight)
    g_h = _segsum(h0, node_in, dest, edge_weight)
    bnd0sl = _banded(hm0, sl)

    rel1 = _relstack(g_sl, bnd0sl, sl,
                     rs1_Wr, rs1_Ws, rs1_gamma, rs1_beta, rewired=True)
    q1, k1 = _qk(rel1, sc1_Wq, sc1_Wk)
    hm1 = _attn(q1, k1, occ)
    bnd1h = _banded(hm1, h0)
    hidden, gsum = _gearnet(g_h, bnd1h, h0, gn1_Wr, gn1_Ws)

    return gsum.reshape(D), hidden
